# Initial kernel scaffold; baseline (speedup 1.0000x reference)
import jax
import jax.numpy as jnp
from jax.experimental import pallas as pl


def kernel(x, edge_index):
    raise NotImplementedError("write your pallas kernel here")


# speedup vs baseline: 4.1102x; 4.1102x over previous
import functools

import jax
import jax.numpy as jnp
from jax import lax
from jax.experimental import pallas as pl
from jax.experimental.pallas import tpu as pltpu
from jax.experimental.pallas import tpu_sc as plsc

N_NODES = 10000
D = 128
N_PAD = 10240
E_PAD = 327680
CHUNKS = 80
CW = 128
ROWS_PER_TILE = N_PAD // 16

_mesh = plsc.VectorSubcoreMesh(core_axis_name="c", subcore_axis_name="s")


@functools.partial(
    pl.kernel,
    mesh=_mesh,
    out_type=jax.ShapeDtypeStruct((2, N_PAD), jnp.float32),
    scratch_types=[
        pltpu.VMEM((CHUNKS, CW), jnp.int32),
        pltpu.VMEM((CW,), jnp.float32),
        pltpu.VMEM_SHARED((N_PAD,), jnp.float32),
    ],
)
def _deg_kernel(dst_hbm, zeros1_hbm, deg_out, idx_v, ones_v, deg_sh):
    c = lax.axis_index("c")
    s = lax.axis_index("s")
    w = s * 2 + c
    pltpu.sync_copy(dst_hbm.at[w], idx_v)
    for k in range(CW // 16):
        ones_v[pl.ds(k * 16, 16)] = jnp.ones((16,), jnp.float32)
    sl = pl.ds(s * ROWS_PER_TILE, ROWS_PER_TILE)
    pltpu.sync_copy(zeros1_hbm.at[sl], deg_sh.at[sl])
    plsc.subcore_barrier()

    def body(j, carry):
        pltpu.sync_copy(ones_v, deg_sh.at[idx_v.at[j]], add=True)
        return carry

    lax.fori_loop(0, CHUNKS, body, 0)
    plsc.subcore_barrier()
    pltpu.sync_copy(deg_sh.at[sl], deg_out.at[c, sl])


@functools.partial(
    pl.kernel,
    mesh=_mesh,
    out_type=jax.ShapeDtypeStruct((2, N_PAD, D), jnp.float32),
    scratch_types=[
        pltpu.VMEM((CHUNKS, CW), jnp.int32),
        pltpu.VMEM((CHUNKS, CW), jnp.int32),
        pltpu.VMEM((CW, D), jnp.float32),
        pltpu.VMEM_SHARED((N_PAD, D), jnp.float32),
        pltpu.SemaphoreType.DMA,
    ],
)
def _agg_kernel(src_hbm, dst_hbm, xn_hbm, zeros2_hbm, agg_out,
                sidx_v, didx_v, rows_v, agg_sh, sem):
    c = lax.axis_index("c")
    s = lax.axis_index("s")
    w = s * 2 + c
    pltpu.sync_copy(src_hbm.at[w], sidx_v)
    pltpu.sync_copy(dst_hbm.at[w], didx_v)
    sl = pl.ds(s * ROWS_PER_TILE, ROWS_PER_TILE)
    pltpu.sync_copy(zeros2_hbm.at[sl], agg_sh.at[sl])
    plsc.subcore_barrier()

    def body(j, carry):
        pltpu.async_copy(xn_hbm.at[sidx_v.at[j]], rows_v, sem).wait()
        pltpu.sync_copy(rows_v, agg_sh.at[didx_v.at[j]], add=True)
        return carry

    lax.fori_loop(0, CHUNKS, body, 0)
    plsc.subcore_barrier()
    pltpu.sync_copy(agg_sh.at[sl], agg_out.at[c, sl])


def _xnorm_body(deg_ref, x_ref, o_ref):
    deg = deg_ref[0] + deg_ref[1]
    norm = lax.rsqrt(1.0 + deg)
    o_ref[...] = x_ref[...] * norm[:, None]


def _xnorm(deg_p, x_pad):
    blk = N_PAD // 8
    return pl.pallas_call(
        _xnorm_body,
        grid=(8,),
        in_specs=[
            pl.BlockSpec((2, blk), lambda i: (0, i)),
            pl.BlockSpec((blk, D), lambda i: (i, 0)),
        ],
        out_specs=pl.BlockSpec((blk, D), lambda i: (i, 0)),
        out_shape=jax.ShapeDtypeStruct((N_PAD, D), jnp.float32),
    )(deg_p, x_pad)


def _final_body(agg_ref, xn_ref, deg_ref, o_ref):
    deg = deg_ref[0] + deg_ref[1]
    norm = lax.rsqrt(1.0 + deg)
    agg = agg_ref[0] + agg_ref[1]
    o_ref[...] = (agg + xn_ref[...]) * norm[:, None]


def _final(agg_p, xn, deg_p):
    blk = N_PAD // 8
    return pl.pallas_call(
        _final_body,
        grid=(8,),
        in_specs=[
            pl.BlockSpec((2, blk, D), lambda i: (0, i, 0)),
            pl.BlockSpec((blk, D), lambda i: (i, 0)),
            pl.BlockSpec((2, blk), lambda i: (0, i)),
        ],
        out_specs=pl.BlockSpec((blk, D), lambda i: (i, 0)),
        out_shape=jax.ShapeDtypeStruct((N_PAD, D), jnp.float32),
    )(agg_p, xn, deg_p)


def kernel(x, edge_index):
    ei = edge_index.astype(jnp.int32)
    n_edges = ei.shape[1]
    pad = E_PAD - n_edges
    src = jnp.concatenate(
        [ei[0], jnp.full((pad,), N_NODES, jnp.int32)]).reshape(32, CHUNKS, CW)
    dst = jnp.concatenate(
        [ei[1], jnp.full((pad,), N_NODES, jnp.int32)]).reshape(32, CHUNKS, CW)
    x_pad = jnp.pad(x, ((0, N_PAD - x.shape[0]), (0, 0)))
    zeros1 = jnp.zeros((N_PAD,), jnp.float32)
    zeros2 = jnp.zeros((N_PAD, D), jnp.float32)

    deg_p = _deg_kernel(dst, zeros1)
    xn = _xnorm(deg_p, x_pad)
    agg_p = _agg_kernel(src, dst, xn, zeros2)
    out = _final(agg_p, xn, deg_p)
    return out[:N_NODES]


# speedup vs baseline: 4.3965x; 1.0697x over previous
import functools

import jax
import jax.numpy as jnp
from jax import lax
from jax.experimental import pallas as pl
from jax.experimental.pallas import tpu as pltpu
from jax.experimental.pallas import tpu_sc as plsc

N_NODES = 10000
D = 128
N_PAD = 10240
E_PAD = 327680
CHUNKS = 80
CW = 128
ROWS_PER_TILE = N_PAD // 16

_mesh = plsc.VectorSubcoreMesh(core_axis_name="c", subcore_axis_name="s")


@functools.partial(
    pl.kernel,
    mesh=_mesh,
    out_type=jax.ShapeDtypeStruct((2, N_PAD), jnp.float32),
    scratch_types=[
        pltpu.VMEM((CHUNKS, CW), jnp.int32),
        pltpu.VMEM((CW,), jnp.float32),
        pltpu.VMEM_SHARED((N_PAD,), jnp.float32),
    ],
)
def _deg_kernel(dst_hbm, zeros1_hbm, deg_out, idx_v, ones_v, deg_sh):
    c = lax.axis_index("c")
    s = lax.axis_index("s")
    w = s * 2 + c
    pltpu.sync_copy(dst_hbm.at[w], idx_v)
    for k in range(CW // 16):
        ones_v[pl.ds(k * 16, 16)] = jnp.ones((16,), jnp.float32)
    sl = pl.ds(s * ROWS_PER_TILE, ROWS_PER_TILE)
    pltpu.sync_copy(zeros1_hbm.at[sl], deg_sh.at[sl])
    plsc.subcore_barrier()

    def body(j, carry):
        pltpu.sync_copy(ones_v, deg_sh.at[idx_v.at[j]], add=True)
        return carry

    lax.fori_loop(0, CHUNKS, body, 0)
    plsc.subcore_barrier()
    pltpu.sync_copy(deg_sh.at[sl], deg_out.at[c, sl])


@functools.partial(
    pl.kernel,
    mesh=_mesh,
    out_type=jax.ShapeDtypeStruct((2, N_PAD, D), jnp.float32),
    scratch_types=[
        pltpu.VMEM((CHUNKS // 5, CW), jnp.int32),
        pltpu.VMEM((CHUNKS // 5, CW), jnp.int32),
        pltpu.VMEM((CW, D), jnp.float32),
        pltpu.VMEM((CW, D), jnp.float32),
        pltpu.VMEM_SHARED((N_PAD, D), jnp.float32),
        pltpu.SemaphoreType.DMA,
        pltpu.SemaphoreType.DMA,
    ],
)
def _agg_kernel(src_hbm, dst_hbm, xn_hbm, zeros2_hbm, agg_out,
                sidx_v, didx_v, rows_a, rows_b, agg_sh, sem_a, sem_b):
    c = lax.axis_index("c")
    s = lax.axis_index("s")
    w = s * 2 + c
    sl = pl.ds(s * ROWS_PER_TILE, ROWS_PER_TILE)
    pltpu.sync_copy(zeros2_hbm.at[sl], agg_sh.at[sl])
    plsc.subcore_barrier()

    gch = CHUNKS // 5
    for g in range(5):
        pltpu.sync_copy(src_hbm.at[w, pl.ds(g * gch, gch)], sidx_v)
        pltpu.sync_copy(dst_hbm.at[w, pl.ds(g * gch, gch)], didx_v)
        pltpu.async_copy(xn_hbm.at[sidx_v.at[0]], rows_a, sem_a)

        def body(i, carry):
            j0 = 2 * i
            j1 = j0 + 1
            pltpu.make_async_copy(
                xn_hbm.at[sidx_v.at[j0]], rows_a, sem_a).wait()
            pltpu.async_copy(xn_hbm.at[sidx_v.at[j1]], rows_b, sem_b)
            pltpu.sync_copy(rows_a, agg_sh.at[didx_v.at[j0]], add=True)
            pltpu.make_async_copy(
                xn_hbm.at[sidx_v.at[j1]], rows_b, sem_b).wait()

            @pl.when(i < gch // 2 - 1)
            def _():
                pltpu.async_copy(xn_hbm.at[sidx_v.at[j0 + 2]], rows_a, sem_a)

            pltpu.sync_copy(rows_b, agg_sh.at[didx_v.at[j1]], add=True)
            return carry

        lax.fori_loop(0, gch // 2, body, 0)
    plsc.subcore_barrier()
    pltpu.sync_copy(agg_sh.at[sl], agg_out.at[c, sl])


def _xnorm_body(deg_ref, x_ref, o_ref):
    deg = deg_ref[0] + deg_ref[1]
    norm = lax.rsqrt(1.0 + deg)
    o_ref[...] = x_ref[...] * norm[:, None]


def _xnorm(deg_p, x_pad):
    blk = N_PAD // 8
    return pl.pallas_call(
        _xnorm_body,
        grid=(8,),
        in_specs=[
            pl.BlockSpec((2, blk), lambda i: (0, i)),
            pl.BlockSpec((blk, D), lambda i: (i, 0)),
        ],
        out_specs=pl.BlockSpec((blk, D), lambda i: (i, 0)),
        out_shape=jax.ShapeDtypeStruct((N_PAD, D), jnp.float32),
    )(deg_p, x_pad)


def _final_body(agg_ref, xn_ref, deg_ref, o_ref):
    deg = deg_ref[0] + deg_ref[1]
    norm = lax.rsqrt(1.0 + deg)
    agg = agg_ref[0] + agg_ref[1]
    o_ref[...] = (agg + xn_ref[...]) * norm[:, None]


def _final(agg_p, xn, deg_p):
    blk = N_PAD // 8
    return pl.pallas_call(
        _final_body,
        grid=(8,),
        in_specs=[
            pl.BlockSpec((2, blk, D), lambda i: (0, i, 0)),
            pl.BlockSpec((blk, D), lambda i: (i, 0)),
            pl.BlockSpec((2, blk), lambda i: (0, i)),
        ],
        out_specs=pl.BlockSpec((blk, D), lambda i: (i, 0)),
        out_shape=jax.ShapeDtypeStruct((N_PAD, D), jnp.float32),
    )(agg_p, xn, deg_p)


def kernel(x, edge_index):
    ei = edge_index.astype(jnp.int32)
    n_edges = ei.shape[1]
    pad = E_PAD - n_edges
    src = jnp.concatenate(
        [ei[0], jnp.full((pad,), N_NODES, jnp.int32)]).reshape(32, CHUNKS, CW)
    dst = jnp.concatenate(
        [ei[1], jnp.full((pad,), N_NODES, jnp.int32)]).reshape(32, CHUNKS, CW)
    x_pad = jnp.pad(x, ((0, N_PAD - x.shape[0]), (0, 0)))
    zeros1 = jnp.zeros((N_PAD,), jnp.float32)
    zeros2 = jnp.zeros((N_PAD, D), jnp.float32)

    deg_p = _deg_kernel(dst, zeros1)
    xn = _xnorm(deg_p, x_pad)
    agg_p = _agg_kernel(src, dst, xn, zeros2)
    out = _final(agg_p, xn, deg_p)
    return out[:N_NODES]


# speedup vs baseline: 4.6269x; 1.0524x over previous
import functools

import jax
import jax.numpy as jnp
from jax import lax
from jax.experimental import pallas as pl
from jax.experimental.pallas import tpu as pltpu
from jax.experimental.pallas import tpu_sc as plsc

N_NODES = 10000
D = 128
N_PAD = 10240
E_PAD = 327680
CHUNKS = 80
CW = 128
CPT0 = 120
CPT1 = 40
GCH = 8
ROWS_PER_TILE = N_PAD // 16

_mesh = plsc.VectorSubcoreMesh(core_axis_name="c", subcore_axis_name="s")


@functools.partial(
    pl.kernel,
    mesh=_mesh,
    out_type=jax.ShapeDtypeStruct((2, N_PAD), jnp.float32),
    scratch_types=[
        pltpu.VMEM((CHUNKS, CW), jnp.int32),
        pltpu.VMEM((CW,), jnp.float32),
        pltpu.VMEM_SHARED((N_PAD,), jnp.float32),
    ],
)
def _deg_kernel(dst_hbm, zeros1_hbm, deg_out, idx_v, ones_v, deg_sh):
    c = lax.axis_index("c")
    s = lax.axis_index("s")
    w = s * 2 + c
    pltpu.sync_copy(dst_hbm.at[pl.ds(w * CHUNKS, CHUNKS)], idx_v)
    for k in range(CW // 16):
        ones_v[pl.ds(k * 16, 16)] = jnp.ones((16,), jnp.float32)
    sl = pl.ds(s * ROWS_PER_TILE, ROWS_PER_TILE)
    pltpu.sync_copy(zeros1_hbm.at[sl], deg_sh.at[sl])
    plsc.subcore_barrier()

    def body(j, carry):
        pltpu.sync_copy(ones_v, deg_sh.at[idx_v.at[j]], add=True)
        return carry

    lax.fori_loop(0, CHUNKS, body, 0)
    plsc.subcore_barrier()
    pltpu.sync_copy(deg_sh.at[sl], deg_out.at[c, sl])


@functools.partial(
    pl.kernel,
    mesh=_mesh,
    out_type=jax.ShapeDtypeStruct((2, N_PAD, D), jnp.float32),
    scratch_types=[
        pltpu.VMEM((GCH, CW), jnp.int32),
        pltpu.VMEM((GCH, CW), jnp.int32),
        pltpu.VMEM((CW, D), jnp.float32),
        pltpu.VMEM((CW, D), jnp.float32),
        pltpu.VMEM_SHARED((N_PAD, D), jnp.float32),
        pltpu.SemaphoreType.DMA,
        pltpu.SemaphoreType.DMA,
    ],
)
def _agg_kernel(src_hbm, dst_hbm, xn_hbm, zeros2_hbm, agg_out,
                sidx_v, didx_v, rows_a, rows_b, agg_sh, sem_a, sem_b):
    c = lax.axis_index("c")
    s = lax.axis_index("s")
    sl = pl.ds(s * ROWS_PER_TILE, ROWS_PER_TILE)
    pltpu.sync_copy(zeros2_hbm.at[sl], agg_sh.at[sl])
    plsc.subcore_barrier()

    nch = jnp.where(c == 0, CPT0, CPT1)
    base = jnp.where(c == 0, s * CPT0, 16 * CPT0 + s * CPT1)

    def group_body(g, carry):
        cb = base + g * GCH
        pltpu.sync_copy(src_hbm.at[pl.ds(cb, GCH)], sidx_v)
        pltpu.sync_copy(dst_hbm.at[pl.ds(cb, GCH)], didx_v)
        pltpu.async_copy(xn_hbm.at[sidx_v.at[0]], rows_a, sem_a)

        def body(i, carry2):
            j0 = 2 * i
            j1 = j0 + 1
            pltpu.make_async_copy(
                xn_hbm.at[sidx_v.at[j0]], rows_a, sem_a).wait()
            pltpu.async_copy(xn_hbm.at[sidx_v.at[j1]], rows_b, sem_b)
            pltpu.sync_copy(rows_a, agg_sh.at[didx_v.at[j0]], add=True)
            pltpu.make_async_copy(
                xn_hbm.at[sidx_v.at[j1]], rows_b, sem_b).wait()

            @pl.when(i < GCH // 2 - 1)
            def _():
                pltpu.async_copy(xn_hbm.at[sidx_v.at[j0 + 2]], rows_a, sem_a)

            pltpu.sync_copy(rows_b, agg_sh.at[didx_v.at[j1]], add=True)
            return carry2

        lax.fori_loop(0, GCH // 2, body, 0)
        return carry

    lax.fori_loop(0, nch // GCH, group_body, 0)
    plsc.subcore_barrier()
    pltpu.sync_copy(agg_sh.at[sl], agg_out.at[c, sl])


def _xnorm_body(deg_ref, x_ref, o_ref):
    deg = deg_ref[0] + deg_ref[1]
    norm = lax.rsqrt(1.0 + deg)
    o_ref[...] = x_ref[...] * norm[:, None]


def _xnorm(deg_p, x_pad):
    blk = N_PAD // 8
    return pl.pallas_call(
        _xnorm_body,
        grid=(8,),
        in_specs=[
            pl.BlockSpec((2, blk), lambda i: (0, i)),
            pl.BlockSpec((blk, D), lambda i: (i, 0)),
        ],
        out_specs=pl.BlockSpec((blk, D), lambda i: (i, 0)),
        out_shape=jax.ShapeDtypeStruct((N_PAD, D), jnp.float32),
    )(deg_p, x_pad)


def _final_body(agg_ref, xn_ref, deg_ref, o_ref):
    deg = deg_ref[0] + deg_ref[1]
    norm = lax.rsqrt(1.0 + deg)
    agg = agg_ref[0] + agg_ref[1]
    o_ref[...] = (agg + xn_ref[...]) * norm[:, None]


def _final(agg_p, xn, deg_p):
    blk = N_PAD // 8
    return pl.pallas_call(
        _final_body,
        grid=(8,),
        in_specs=[
            pl.BlockSpec((2, blk, D), lambda i: (0, i, 0)),
            pl.BlockSpec((blk, D), lambda i: (i, 0)),
            pl.BlockSpec((2, blk), lambda i: (0, i)),
        ],
        out_specs=pl.BlockSpec((blk, D), lambda i: (i, 0)),
        out_shape=jax.ShapeDtypeStruct((N_PAD, D), jnp.float32),
    )(agg_p, xn, deg_p)


def kernel(x, edge_index):
    ei = edge_index.astype(jnp.int32)
    n_edges = ei.shape[1]
    pad = E_PAD - n_edges
    src = jnp.concatenate(
        [ei[0], jnp.full((pad,), N_NODES, jnp.int32)]).reshape(-1, CW)
    dst = jnp.concatenate(
        [ei[1], jnp.full((pad,), N_NODES, jnp.int32)]).reshape(-1, CW)
    x_pad = jnp.pad(x, ((0, N_PAD - x.shape[0]), (0, 0)))
    zeros1 = jnp.zeros((N_PAD,), jnp.float32)
    zeros2 = jnp.zeros((N_PAD, D), jnp.float32)

    deg_p = _deg_kernel(dst, zeros1)
    xn = _xnorm(deg_p, x_pad)
    agg_p = _agg_kernel(src, dst, xn, zeros2)
    out = _final(agg_p, xn, deg_p)
    return out[:N_NODES]
